# baseline (speedup 1.0000x reference)
import functools

import jax
import jax.numpy as jnp
from jax import lax
from jax.experimental import pallas as pl
from jax.experimental.pallas import tpu as pltpu
from jax.experimental.pallas import tpu_sc as plsc

N = 10000
E = 320000
D = 128

NC = 2
NS = 16
NW = NC * NS

K = 80
EPT = E // NW
NCHUNK = EPT // K
NPAD = 10112
ROWS_PT = NPAD // NS
ZR = 40

DK = 125
DEPT = E // NS
DNCHUNK = DEPT // DK
SLAB = NPAD // NS

_mesh = plsc.VectorSubcoreMesh(
    core_axis_name="c", subcore_axis_name="s", num_cores=NC, num_subcores=NS
)


@functools.partial(
    pl.kernel,
    out_type=jax.ShapeDtypeStruct((NC * NPAD,), jnp.float32),
    mesh=_mesh,
    scratch_types=[
        pltpu.VMEM((DNCHUNK, DK), jnp.int32),
        pltpu.VMEM((128,), jnp.float32),
        pltpu.VMEM((640,), jnp.float32),
        pltpu.VMEM_SHARED((NPAD,), jnp.float32),
    ],
)
def _deg_kernel(eidx_hbm, degp_hbm, idx_v, ones_v, z_v, cnt_sh):
    c = lax.axis_index("c")
    s = lax.axis_index("s")

    pltpu.sync_copy(eidx_hbm.at[c, s], idx_v)

    def fill_ones(i, carry):
        ones_v[pl.ds(i * 16, 16)] = jnp.ones((16,), jnp.float32)
        return carry

    lax.fori_loop(0, 128 // 16, fill_ones, 0)

    def fill_zeros(i, carry):
        z_v[pl.ds(i * 16, 16)] = jnp.zeros((16,), jnp.float32)
        return carry

    lax.fori_loop(0, 640 // 16, fill_zeros, 0)

    sl = pl.ds(s * SLAB, SLAB)
    pltpu.sync_copy(z_v.at[pl.ds(0, SLAB)], cnt_sh.at[sl])
    plsc.subcore_barrier()

    def body(j, carry):
        pltpu.sync_copy(ones_v.at[pl.ds(0, DK)], cnt_sh.at[idx_v.at[j]],
                        add=True)
        return carry

    lax.fori_loop(0, DNCHUNK, body, 0)

    plsc.subcore_barrier()
    pltpu.sync_copy(cnt_sh.at[sl], z_v.at[pl.ds(0, SLAB)])
    pltpu.sync_copy(z_v.at[pl.ds(0, SLAB)],
                    degp_hbm.at[pl.ds(c * NPAD + s * SLAB, SLAB)])


@functools.partial(
    pl.kernel,
    out_type=jax.ShapeDtypeStruct((NC, NPAD, D), jnp.float32),
    mesh=_mesh,
    scratch_types=[
        pltpu.VMEM((EPT,), jnp.int32),
        pltpu.VMEM((NCHUNK, K), jnp.int32),
        pltpu.VMEM((2, K, D), jnp.float32),
        pltpu.VMEM_SHARED((NPAD, D), jnp.float32),
        pltpu.SemaphoreType.DMA,
        pltpu.SemaphoreType.DMA,
        pltpu.SemaphoreType.DMA,
        pltpu.SemaphoreType.DMA,
    ],
)
def _scat_kernel(h_hbm, src_hbm, dst_hbm, aggp_hbm, src_v, dst_v, rows_v,
                 acc_sh, gsem0, gsem1, ssem0, ssem1):
    c = lax.axis_index("c")
    s = lax.axis_index("s")
    w = c * NS + s

    pltpu.sync_copy(src_hbm.at[w], src_v)
    pltpu.sync_copy(dst_hbm.at[w], dst_v)

    def fill_zeros(i, carry):
        rows_v[0, i // 8, pl.ds((i % 8) * 16, 16)] = jnp.zeros((16,),
                                                               jnp.float32)
        return carry

    lax.fori_loop(0, ZR * D // 16, fill_zeros, 0)
    for i in range(ROWS_PT // ZR):
        pltpu.sync_copy(rows_v.at[0, pl.ds(0, ZR)],
                        acc_sh.at[pl.ds(s * ROWS_PT + i * ZR, ZR)])
    _rem = ROWS_PT - (ROWS_PT // ZR) * ZR
    if _rem:
        pltpu.sync_copy(
            rows_v.at[0, pl.ds(0, _rem)],
            acc_sh.at[pl.ds(s * ROWS_PT + (ROWS_PT // ZR) * ZR, _rem)])
    plsc.subcore_barrier()

    def src_at(j):
        return src_v.at[pl.ds(j * K, K)]

    def gather(j, b, sem):
        return pltpu.async_copy(h_hbm.at[src_at(j)], rows_v.at[b], sem)

    def scatter(j, b, sem):
        return pltpu.async_copy(rows_v.at[b], acc_sh.at[dst_v.at[j]], sem,
                                add=True)

    def gwait(j, b, sem):
        pltpu.make_async_copy(h_hbm.at[src_at(j)], rows_v.at[b], sem).wait()

    def swait(j, b, sem):
        pltpu.make_async_copy(rows_v.at[b], acc_sh.at[dst_v.at[j]],
                              sem).wait()

    gather(0, 0, gsem0)
    gather(1, 1, gsem1)

    def body(i, carry):
        j0 = i * 2
        gwait(j0, 0, gsem0)
        scatter(j0, 0, ssem0)
        gwait(j0 + 1, 1, gsem1)
        scatter(j0 + 1, 1, ssem1)

        @pl.when(j0 + 2 < NCHUNK)
        def _():
            swait(j0, 0, ssem0)
            gather(j0 + 2, 0, gsem0)

        @pl.when(j0 + 3 < NCHUNK)
        def _():
            swait(j0 + 1, 1, ssem1)
            gather(j0 + 3, 1, gsem1)

        return carry

    lax.fori_loop(0, NCHUNK // 2, body, 0)

    if NCHUNK % 2:
        jl = NCHUNK - 1
        gwait(jl, 0, gsem0)
        scatter(jl, 0, ssem0)
        swait(jl, 0, ssem0)
        swait(jl - 1, 1, ssem1)
    else:
        swait(NCHUNK - 2, 0, ssem0)
        swait(NCHUNK - 1, 1, ssem1)

    plsc.subcore_barrier()
    rows = pl.ds(s * ROWS_PT, ROWS_PT)
    pltpu.sync_copy(acc_sh.at[rows], aggp_hbm.at[c, rows])


BR = 1000
GR = N // BR


def _norm_col(deg_blk):
    return lax.rsqrt(jnp.maximum(deg_blk, 1.0))


def _mm1_body(feats_r, degs_r, w_r, o_r):
    onorm = _norm_col(degs_r[...])
    x = feats_r[...] * onorm
    o_r[...] = jnp.dot(x, w_r[...], preferred_element_type=jnp.float32,
                       precision=lax.Precision.HIGHEST)


def _mm2_body(aggp_r, degs_r, degd_r, w_r, b_r, o_r):
    a = aggp_r[0] + aggp_r[1]
    inorm = _norm_col(degd_r[...])
    onorm = _norm_col(degs_r[...])
    x = (a * inorm + b_r[...]) * onorm
    o_r[...] = jnp.dot(x, w_r[...], preferred_element_type=jnp.float32,
                       precision=lax.Precision.HIGHEST)


def _fin_body(aggp_r, degd_r, b_r, o_r):
    a = aggp_r[0] + aggp_r[1]
    inorm = _norm_col(degd_r[...])
    o_r[...] = a * inorm + b_r[...]


def _tc_mm1(feats, degsrc_t, w1):
    return pl.pallas_call(
        _mm1_body,
        grid=(GR,),
        in_specs=[
            pl.BlockSpec((BR, D), lambda i: (i, 0)),
            pl.BlockSpec((BR, 1), lambda i: (i, 0)),
            pl.BlockSpec((D, D), lambda i: (0, 0)),
        ],
        out_specs=pl.BlockSpec((BR, D), lambda i: (i, 0)),
        out_shape=jax.ShapeDtypeStruct((N, D), jnp.float32),
    )(feats, degsrc_t, w1)


def _tc_mm2(aggp, degsrc_t, degdst_t, w2, b1):
    return pl.pallas_call(
        _mm2_body,
        grid=(GR,),
        in_specs=[
            pl.BlockSpec((NC, BR, D), lambda i: (0, i, 0)),
            pl.BlockSpec((BR, 1), lambda i: (i, 0)),
            pl.BlockSpec((BR, 1), lambda i: (i, 0)),
            pl.BlockSpec((D, D), lambda i: (0, 0)),
            pl.BlockSpec((1, D), lambda i: (0, 0)),
        ],
        out_specs=pl.BlockSpec((BR, D), lambda i: (i, 0)),
        out_shape=jax.ShapeDtypeStruct((N, D), jnp.float32),
    )(aggp, degsrc_t, degdst_t, w2, b1)


def _tc_fin(aggp, degdst_t, b2):
    return pl.pallas_call(
        _fin_body,
        grid=(GR,),
        in_specs=[
            pl.BlockSpec((NC, BR, D), lambda i: (0, i, 0)),
            pl.BlockSpec((BR, 1), lambda i: (i, 0)),
            pl.BlockSpec((1, D), lambda i: (0, 0)),
        ],
        out_specs=pl.BlockSpec((BR, D), lambda i: (i, 0)),
        out_shape=jax.ShapeDtypeStruct((N, D), jnp.float32),
    )(aggp, degdst_t, b2)


def kernel(feats, edge_index, W1, b1, W2, b2):
    ei = edge_index.astype(jnp.int32)
    eidx = ei.reshape(NC, NS, DNCHUNK, DK)
    srcp = ei[0].reshape(NW, EPT)
    dstp = ei[1].reshape(NW, NCHUNK, K)

    degp = _deg_kernel(eidx).reshape(NC, NPAD)
    deg_t = jnp.transpose(degp)
    degsrc_t = deg_t[:N, 0:1]
    degdst_t = deg_t[:N, 1:2]

    h1 = _tc_mm1(feats, degsrc_t, W1)
    aggp1 = _scat_kernel(h1, srcp, dstp)
    h2 = _tc_mm2(aggp1, degsrc_t, degdst_t, W2, b1.reshape(1, D))
    aggp2 = _scat_kernel(h2, srcp, dstp)
    return _tc_fin(aggp2, degdst_t, b2.reshape(1, D))


# speedup vs baseline: 1.2137x; 1.2137x over previous
import functools

import jax
import jax.numpy as jnp
from jax import lax
from jax.experimental import pallas as pl
from jax.experimental.pallas import tpu as pltpu
from jax.experimental.pallas import tpu_sc as plsc

N = 10000
E = 320000
D = 128

NC = 2
NS = 16
NW = NC * NS

K = 80
EPT = E // NW
NCHUNK = EPT // K
NPAD = 10112
ROWS_PT = NPAD // NS
ZR = 40

DK = 125
DEPT = E // NS
DNCHUNK = DEPT // DK
SLAB = NPAD // NS

_mesh = plsc.VectorSubcoreMesh(
    core_axis_name="c", subcore_axis_name="s", num_cores=NC, num_subcores=NS
)


@functools.partial(
    pl.kernel,
    out_type=jax.ShapeDtypeStruct((NC * NPAD,), jnp.float32),
    mesh=_mesh,
    scratch_types=[
        pltpu.VMEM((DNCHUNK, DK), jnp.int32),
        pltpu.VMEM((128,), jnp.float32),
        pltpu.VMEM((640,), jnp.float32),
        pltpu.VMEM_SHARED((NPAD,), jnp.float32),
    ],
)
def _deg_kernel(eidx_hbm, degp_hbm, idx_v, ones_v, z_v, cnt_sh):
    c = lax.axis_index("c")
    s = lax.axis_index("s")

    pltpu.sync_copy(eidx_hbm.at[c, s], idx_v)

    def fill_ones(i, carry):
        ones_v[pl.ds(i * 16, 16)] = jnp.ones((16,), jnp.float32)
        return carry

    lax.fori_loop(0, 128 // 16, fill_ones, 0)

    def fill_zeros(i, carry):
        z_v[pl.ds(i * 16, 16)] = jnp.zeros((16,), jnp.float32)
        return carry

    lax.fori_loop(0, 640 // 16, fill_zeros, 0)

    sl = pl.ds(s * SLAB, SLAB)
    pltpu.sync_copy(z_v.at[pl.ds(0, SLAB)], cnt_sh.at[sl])
    plsc.subcore_barrier()

    def body(j, carry):
        pltpu.sync_copy(ones_v.at[pl.ds(0, DK)], cnt_sh.at[idx_v.at[j]],
                        add=True)
        return carry

    lax.fori_loop(0, DNCHUNK, body, 0)

    plsc.subcore_barrier()
    pltpu.sync_copy(cnt_sh.at[sl], z_v.at[pl.ds(0, SLAB)])
    pltpu.sync_copy(z_v.at[pl.ds(0, SLAB)],
                    degp_hbm.at[pl.ds(c * NPAD + s * SLAB, SLAB)])


@functools.partial(
    pl.kernel,
    out_type=jax.ShapeDtypeStruct((NC, NPAD, D), jnp.float32),
    mesh=_mesh,
    scratch_types=[
        pltpu.VMEM((EPT,), jnp.int32),
        pltpu.VMEM((NCHUNK, K), jnp.int32),
        pltpu.VMEM((2, K, D), jnp.float32),
        pltpu.VMEM_SHARED((NPAD, D), jnp.float32),
        pltpu.SemaphoreType.DMA,
        pltpu.SemaphoreType.DMA,
        pltpu.SemaphoreType.DMA,
        pltpu.SemaphoreType.DMA,
    ],
)
def _scat_kernel(h_hbm, src_hbm, dst_hbm, aggp_hbm, src_v, dst_v, rows_v,
                 acc_sh, gsem0, gsem1, ssem0, ssem1):
    c = lax.axis_index("c")
    s = lax.axis_index("s")
    w = c * NS + s

    pltpu.sync_copy(src_hbm.at[w], src_v)
    pltpu.sync_copy(dst_hbm.at[w], dst_v)

    def fill_zeros(i, carry):
        rows_v[0, i // 8, pl.ds((i % 8) * 16, 16)] = jnp.zeros((16,),
                                                               jnp.float32)
        return carry

    lax.fori_loop(0, ZR * D // 16, fill_zeros, 0)
    for i in range(ROWS_PT // ZR):
        pltpu.sync_copy(rows_v.at[0, pl.ds(0, ZR)],
                        acc_sh.at[pl.ds(s * ROWS_PT + i * ZR, ZR)])
    _rem = ROWS_PT - (ROWS_PT // ZR) * ZR
    if _rem:
        pltpu.sync_copy(
            rows_v.at[0, pl.ds(0, _rem)],
            acc_sh.at[pl.ds(s * ROWS_PT + (ROWS_PT // ZR) * ZR, _rem)])
    plsc.subcore_barrier()

    def src_at(j):
        return src_v.at[pl.ds(j * K, K)]

    pltpu.async_copy(h_hbm.at[src_at(0)], rows_v.at[0], gsem0)
    pltpu.async_copy(h_hbm.at[src_at(1)], rows_v.at[1], gsem1)

    def body(i, carry):
        j0 = i * 2
        pltpu.make_async_copy(h_hbm.at[src_at(j0)], rows_v.at[0],
                              gsem0).wait()
        pltpu.sync_copy(rows_v.at[0], acc_sh.at[dst_v.at[j0]], add=True)

        @pl.when(j0 + 2 < NCHUNK)
        def _():
            pltpu.async_copy(h_hbm.at[src_at(j0 + 2)], rows_v.at[0], gsem0)

        pltpu.make_async_copy(h_hbm.at[src_at(j0 + 1)], rows_v.at[1],
                              gsem1).wait()
        pltpu.sync_copy(rows_v.at[1], acc_sh.at[dst_v.at[j0 + 1]], add=True)

        @pl.when(j0 + 3 < NCHUNK)
        def _():
            pltpu.async_copy(h_hbm.at[src_at(j0 + 3)], rows_v.at[1], gsem1)

        return carry

    lax.fori_loop(0, NCHUNK // 2, body, 0)

    if NCHUNK % 2:
        jl = NCHUNK - 1
        pltpu.make_async_copy(h_hbm.at[src_at(jl)], rows_v.at[0],
                              gsem0).wait()
        pltpu.sync_copy(rows_v.at[0], acc_sh.at[dst_v.at[jl]], add=True)

    plsc.subcore_barrier()
    rows = pl.ds(s * ROWS_PT, ROWS_PT)
    pltpu.sync_copy(acc_sh.at[rows], aggp_hbm.at[c, rows])


BR = 1000
GR = N // BR


def _norm_col(deg_blk):
    return lax.rsqrt(jnp.maximum(deg_blk, 1.0))


def _mm1_body(feats_r, degs_r, w_r, o_r):
    onorm = _norm_col(degs_r[...])
    x = feats_r[...] * onorm
    o_r[...] = jnp.dot(x, w_r[...], preferred_element_type=jnp.float32,
                       precision=lax.Precision.HIGHEST)


def _mm2_body(aggp_r, degs_r, degd_r, w_r, b_r, o_r):
    a = aggp_r[0] + aggp_r[1]
    inorm = _norm_col(degd_r[...])
    onorm = _norm_col(degs_r[...])
    x = (a * inorm + b_r[...]) * onorm
    o_r[...] = jnp.dot(x, w_r[...], preferred_element_type=jnp.float32,
                       precision=lax.Precision.HIGHEST)


def _fin_body(aggp_r, degd_r, b_r, o_r):
    a = aggp_r[0] + aggp_r[1]
    inorm = _norm_col(degd_r[...])
    o_r[...] = a * inorm + b_r[...]


def _tc_mm1(feats, degsrc_t, w1):
    return pl.pallas_call(
        _mm1_body,
        grid=(GR,),
        in_specs=[
            pl.BlockSpec((BR, D), lambda i: (i, 0)),
            pl.BlockSpec((BR, 1), lambda i: (i, 0)),
            pl.BlockSpec((D, D), lambda i: (0, 0)),
        ],
        out_specs=pl.BlockSpec((BR, D), lambda i: (i, 0)),
        out_shape=jax.ShapeDtypeStruct((N, D), jnp.float32),
    )(feats, degsrc_t, w1)


def _tc_mm2(aggp, degsrc_t, degdst_t, w2, b1):
    return pl.pallas_call(
        _mm2_body,
        grid=(GR,),
        in_specs=[
            pl.BlockSpec((NC, BR, D), lambda i: (0, i, 0)),
            pl.BlockSpec((BR, 1), lambda i: (i, 0)),
            pl.BlockSpec((BR, 1), lambda i: (i, 0)),
            pl.BlockSpec((D, D), lambda i: (0, 0)),
            pl.BlockSpec((1, D), lambda i: (0, 0)),
        ],
        out_specs=pl.BlockSpec((BR, D), lambda i: (i, 0)),
        out_shape=jax.ShapeDtypeStruct((N, D), jnp.float32),
    )(aggp, degsrc_t, degdst_t, w2, b1)


def _tc_fin(aggp, degdst_t, b2):
    return pl.pallas_call(
        _fin_body,
        grid=(GR,),
        in_specs=[
            pl.BlockSpec((NC, BR, D), lambda i: (0, i, 0)),
            pl.BlockSpec((BR, 1), lambda i: (i, 0)),
            pl.BlockSpec((1, D), lambda i: (0, 0)),
        ],
        out_specs=pl.BlockSpec((BR, D), lambda i: (i, 0)),
        out_shape=jax.ShapeDtypeStruct((N, D), jnp.float32),
    )(aggp, degdst_t, b2)


def kernel(feats, edge_index, W1, b1, W2, b2):
    ei = edge_index.astype(jnp.int32)
    eidx = ei.reshape(NC, NS, DNCHUNK, DK)
    srcp = ei[0].reshape(NW, EPT)
    dstp = ei[1].reshape(NW, NCHUNK, K)

    degp = _deg_kernel(eidx).reshape(NC, NPAD)
    deg_t = jnp.transpose(degp)
    degsrc_t = deg_t[:N, 0:1]
    degdst_t = deg_t[:N, 1:2]

    h1 = _tc_mm1(feats, degsrc_t, W1)
    aggp1 = _scat_kernel(h1, srcp, dstp)
    h2 = _tc_mm2(aggp1, degsrc_t, degdst_t, W2, b1.reshape(1, D))
    aggp2 = _scat_kernel(h2, srcp, dstp)
    return _tc_fin(aggp2, degdst_t, b2.reshape(1, D))


# speedup vs baseline: 1.2697x; 1.0461x over previous
import functools

import jax
import jax.numpy as jnp
from jax import lax
from jax.experimental import pallas as pl
from jax.experimental.pallas import tpu as pltpu
from jax.experimental.pallas import tpu_sc as plsc

N = 10000
E = 320000
D = 128

NC = 2
NS = 16
NW = NC * NS

K = 80
EPT = E // NW
NCHUNK = EPT // K
NPAD = 10112
ROWS_PT = NPAD // NS
ZR = 40

DK = 125
DEPT = E // NS
DNCHUNK = DEPT // DK
SLAB = NPAD // NS

_mesh = plsc.VectorSubcoreMesh(
    core_axis_name="c", subcore_axis_name="s", num_cores=NC, num_subcores=NS
)


@functools.partial(
    pl.kernel,
    out_type=jax.ShapeDtypeStruct((NC * NPAD,), jnp.float32),
    mesh=_mesh,
    scratch_types=[
        pltpu.VMEM((DNCHUNK, DK), jnp.int32),
        pltpu.VMEM((128,), jnp.float32),
        pltpu.VMEM((640,), jnp.float32),
        pltpu.VMEM_SHARED((NPAD,), jnp.float32),
    ],
)
def _deg_kernel(eidx_hbm, degp_hbm, idx_v, ones_v, z_v, cnt_sh):
    c = lax.axis_index("c")
    s = lax.axis_index("s")

    pltpu.sync_copy(eidx_hbm.at[c, s], idx_v)

    def fill_ones(i, carry):
        ones_v[pl.ds(i * 16, 16)] = jnp.ones((16,), jnp.float32)
        return carry

    lax.fori_loop(0, 128 // 16, fill_ones, 0)

    def fill_zeros(i, carry):
        z_v[pl.ds(i * 16, 16)] = jnp.zeros((16,), jnp.float32)
        return carry

    lax.fori_loop(0, 640 // 16, fill_zeros, 0)

    sl = pl.ds(s * SLAB, SLAB)
    pltpu.sync_copy(z_v.at[pl.ds(0, SLAB)], cnt_sh.at[sl])
    plsc.subcore_barrier()

    def body(j, carry):
        pltpu.sync_copy(ones_v.at[pl.ds(0, DK)], cnt_sh.at[idx_v.at[j]],
                        add=True)
        return carry

    lax.fori_loop(0, DNCHUNK, body, 0)

    plsc.subcore_barrier()
    pltpu.sync_copy(cnt_sh.at[sl], z_v.at[pl.ds(0, SLAB)])
    pltpu.sync_copy(z_v.at[pl.ds(0, SLAB)],
                    degp_hbm.at[pl.ds(c * NPAD + s * SLAB, SLAB)])


@functools.partial(
    pl.kernel,
    out_type=jax.ShapeDtypeStruct((NC, NPAD, D), jnp.float32),
    mesh=_mesh,
    scratch_types=[
        pltpu.VMEM((EPT,), jnp.int32),
        pltpu.VMEM((NCHUNK, K), jnp.int32),
        pltpu.VMEM((2, K, D), jnp.float32),
        pltpu.VMEM_SHARED((NPAD, D), jnp.float32),
        pltpu.SemaphoreType.DMA,
        pltpu.SemaphoreType.DMA,
        pltpu.SemaphoreType.DMA,
        pltpu.SemaphoreType.DMA,
    ],
)
def _scat_kernel(h_hbm, src_hbm, dst_hbm, aggp_hbm, src_v, dst_v, rows_v,
                 acc_sh, gsem0, gsem1, ssem0, ssem1):
    c = lax.axis_index("c")
    s = lax.axis_index("s")
    w = c * NS + s

    pltpu.async_copy(src_hbm.at[w], src_v, ssem0)
    pltpu.async_copy(dst_hbm.at[w], dst_v, ssem1)

    def fill_zeros(i, carry):
        rows_v[0, i // 8, pl.ds((i % 8) * 16, 16)] = jnp.zeros((16,),
                                                               jnp.float32)
        return carry

    lax.fori_loop(0, K * D // 16, fill_zeros, 0)
    for i in range(ROWS_PT // K):
        pltpu.sync_copy(rows_v.at[0],
                        acc_sh.at[pl.ds(s * ROWS_PT + i * K, K)])
    _rem = ROWS_PT - (ROWS_PT // K) * K
    if _rem:
        pltpu.sync_copy(
            rows_v.at[0, pl.ds(0, _rem)],
            acc_sh.at[pl.ds(s * ROWS_PT + (ROWS_PT // K) * K, _rem)])
    pltpu.make_async_copy(src_hbm.at[w], src_v, ssem0).wait()
    pltpu.make_async_copy(dst_hbm.at[w], dst_v, ssem1).wait()
    plsc.subcore_barrier()

    def src_at(j):
        return src_v.at[pl.ds(j * K, K)]

    pltpu.async_copy(h_hbm.at[src_at(0)], rows_v.at[0], gsem0)
    pltpu.async_copy(h_hbm.at[src_at(1)], rows_v.at[1], gsem1)

    def body(i, carry):
        j0 = i * 2
        pltpu.make_async_copy(h_hbm.at[src_at(j0)], rows_v.at[0],
                              gsem0).wait()
        pltpu.sync_copy(rows_v.at[0], acc_sh.at[dst_v.at[j0]], add=True)

        @pl.when(j0 + 2 < NCHUNK)
        def _():
            pltpu.async_copy(h_hbm.at[src_at(j0 + 2)], rows_v.at[0], gsem0)

        pltpu.make_async_copy(h_hbm.at[src_at(j0 + 1)], rows_v.at[1],
                              gsem1).wait()
        pltpu.sync_copy(rows_v.at[1], acc_sh.at[dst_v.at[j0 + 1]], add=True)

        @pl.when(j0 + 3 < NCHUNK)
        def _():
            pltpu.async_copy(h_hbm.at[src_at(j0 + 3)], rows_v.at[1], gsem1)

        return carry

    lax.fori_loop(0, NCHUNK // 2, body, 0)

    if NCHUNK % 2:
        jl = NCHUNK - 1
        pltpu.make_async_copy(h_hbm.at[src_at(jl)], rows_v.at[0],
                              gsem0).wait()
        pltpu.sync_copy(rows_v.at[0], acc_sh.at[dst_v.at[jl]], add=True)

    plsc.subcore_barrier()
    rows = pl.ds(s * ROWS_PT, ROWS_PT)
    pltpu.sync_copy(acc_sh.at[rows], aggp_hbm.at[c, rows])


BR = 2000
GR = N // BR


def _norm_col(deg_blk):
    return lax.rsqrt(jnp.maximum(deg_blk, 1.0))


def _mm1_body(feats_r, degs_r, w_r, o_r):
    onorm = _norm_col(degs_r[...])
    x = feats_r[...] * onorm
    o_r[...] = jnp.dot(x, w_r[...], preferred_element_type=jnp.float32,
                       precision=lax.Precision.HIGHEST)


def _mm2_body(aggp_r, degs_r, degd_r, w_r, b_r, o_r):
    a = aggp_r[0] + aggp_r[1]
    inorm = _norm_col(degd_r[...])
    onorm = _norm_col(degs_r[...])
    x = (a * inorm + b_r[...]) * onorm
    o_r[...] = jnp.dot(x, w_r[...], preferred_element_type=jnp.float32,
                       precision=lax.Precision.HIGHEST)


def _fin_body(aggp_r, degd_r, b_r, o_r):
    a = aggp_r[0] + aggp_r[1]
    inorm = _norm_col(degd_r[...])
    o_r[...] = a * inorm + b_r[...]


def _tc_mm1(feats, degsrc_t, w1):
    return pl.pallas_call(
        _mm1_body,
        grid=(GR,),
        in_specs=[
            pl.BlockSpec((BR, D), lambda i: (i, 0)),
            pl.BlockSpec((BR, 1), lambda i: (i, 0)),
            pl.BlockSpec((D, D), lambda i: (0, 0)),
        ],
        out_specs=pl.BlockSpec((BR, D), lambda i: (i, 0)),
        out_shape=jax.ShapeDtypeStruct((N, D), jnp.float32),
    )(feats, degsrc_t, w1)


def _tc_mm2(aggp, degsrc_t, degdst_t, w2, b1):
    return pl.pallas_call(
        _mm2_body,
        grid=(GR,),
        in_specs=[
            pl.BlockSpec((NC, BR, D), lambda i: (0, i, 0)),
            pl.BlockSpec((BR, 1), lambda i: (i, 0)),
            pl.BlockSpec((BR, 1), lambda i: (i, 0)),
            pl.BlockSpec((D, D), lambda i: (0, 0)),
            pl.BlockSpec((1, D), lambda i: (0, 0)),
        ],
        out_specs=pl.BlockSpec((BR, D), lambda i: (i, 0)),
        out_shape=jax.ShapeDtypeStruct((N, D), jnp.float32),
    )(aggp, degsrc_t, degdst_t, w2, b1)


def _tc_fin(aggp, degdst_t, b2):
    return pl.pallas_call(
        _fin_body,
        grid=(GR,),
        in_specs=[
            pl.BlockSpec((NC, BR, D), lambda i: (0, i, 0)),
            pl.BlockSpec((BR, 1), lambda i: (i, 0)),
            pl.BlockSpec((1, D), lambda i: (0, 0)),
        ],
        out_specs=pl.BlockSpec((BR, D), lambda i: (i, 0)),
        out_shape=jax.ShapeDtypeStruct((N, D), jnp.float32),
    )(aggp, degdst_t, b2)


def kernel(feats, edge_index, W1, b1, W2, b2):
    ei = edge_index.astype(jnp.int32)
    eidx = ei.reshape(NC, NS, DNCHUNK, DK)
    srcp = ei[0].reshape(NW, EPT)
    dstp = ei[1].reshape(NW, NCHUNK, K)

    degp = _deg_kernel(eidx).reshape(NC, NPAD)
    deg_t = jnp.transpose(degp)
    degsrc_t = deg_t[:N, 0:1]
    degdst_t = deg_t[:N, 1:2]

    h1 = _tc_mm1(feats, degsrc_t, W1)
    aggp1 = _scat_kernel(h1, srcp, dstp)
    h2 = _tc_mm2(aggp1, degsrc_t, degdst_t, W2, b1.reshape(1, D))
    aggp2 = _scat_kernel(h2, srcp, dstp)
    return _tc_fin(aggp2, degdst_t, b2.reshape(1, D))
